# baseline (speedup 1.0000x reference)
import functools

import jax
import jax.numpy as jnp
from jax import lax
from jax.experimental import pallas as pl
from jax.experimental.pallas import tpu as pltpu
from jax.experimental.pallas import tpu_sc as plsc

N_PRODUCTS = 100000
N_DIM = 1024
BATCH = 16384
N_LABELS = 3

NC, NS, L = 2, 16, 16
NW = NC * NS
K = 8
NCH = N_DIM // L

SC_ROWS = 8192
TC_ROWS = BATCH - SC_ROWS
TC_BLOCK = 256


def _vrsqrt(x):
    i = lax.bitcast_convert_type(x, jnp.int32)
    magic = jnp.full((L,), 0x5F3759DF, jnp.int32)
    one = jnp.full((L,), 1, jnp.int32)
    y = lax.bitcast_convert_type(
        magic - lax.shift_right_arithmetic(i, one), jnp.float32)
    for _ in range(3):
        y = y * (1.5 - 0.5 * x * y * y)
    return y


def _lane_sum(x, lanes):
    for k in (8, 4, 2, 1):
        x = x + x.at[lanes ^ k].get(mode="promise_in_bounds")
    return x


def _shuf(x, lanes, k):
    return x.at[lanes ^ k].get(mode="promise_in_bounds")


def _merge(a, b, k, lanes):
    m = (lanes & k) != 0
    keep = jnp.where(m, b, a)
    give = jnp.where(m, a, b)
    return keep + _shuf(give, lanes, k)


def _reduce8(vs, lanes):
    for k in (1, 2, 4):
        vs = [_merge(vs[2 * i], vs[2 * i + 1], k, lanes)
              for i in range(len(vs) // 2)]
    z = vs[0]
    return z + _shuf(z, lanes, 8)


_mesh = plsc.VectorSubcoreMesh(
    core_axis_name="c", subcore_axis_name="s", num_cores=NC, num_subcores=NS
)


def _make_sc_fused(nrows):
    rpw = nrows // NW
    nblk = rpw // K

    @functools.partial(
        pl.kernel,
        out_type=jax.ShapeDtypeStruct((N_LABELS * nrows,), jnp.float32),
        mesh=_mesh,
        scratch_types=[
            pltpu.VMEM((rpw,), jnp.int32),
            pltpu.VMEM((3, N_DIM), jnp.float32),
            pltpu.VMEM((3, N_DIM), jnp.float32),
            pltpu.VMEM((2, K, N_DIM), jnp.float32),
            pltpu.VMEM((2, K, N_DIM), jnp.float32),
            pltpu.VMEM((2, K, N_DIM), jnp.float32),
            pltpu.VMEM(((rpw + 8) * N_LABELS,), jnp.float32),
            pltpu.SemaphoreType.DMA,
            pltpu.SemaphoreType.DMA,
        ],
    )
    def sc_fused(qv_hbm, qc_hbm, pvs_hbm, qw_hbm, lab_hbm, idx_hbm, out_hbm,
                 idx_v, qw_v, lab_v, pv_buf, qv_buf, qc_buf, out_v,
                 sem0, sem1):
        wid = lax.axis_index("s") * NC + lax.axis_index("c")
        base = pl.multiple_of(wid * rpw, rpw)
        sems = (sem0, sem1)

        pltpu.sync_copy(idx_hbm.at[pl.ds(base, rpw)], idx_v)
        pltpu.sync_copy(qw_hbm, qw_v)
        pltpu.sync_copy(lab_hbm, lab_v)

        zero = jnp.zeros((L,), jnp.float32)
        lanes = lax.iota(jnp.int32, L)

        def pre_body(v, carry):
            sl = pl.ds(pl.multiple_of(v * L, L), L)
            for j in range(3):
                qw_v[j, sl] = qw_v[j, sl] * 2.0
            l0, l1, l2 = lab_v[0, sl], lab_v[1, sl], lab_v[2, sl]
            a0, a1, a2 = carry
            return (a0 + l0 * l0, a1 + l1 * l1, a2 + l2 * l2)

        la = lax.fori_loop(0, NCH, pre_body, (zero, zero, zero))
        inv_l = [_vrsqrt(_lane_sum(a, lanes)) for a in la]

        def lab_scale(v, c):
            sl = pl.ds(pl.multiple_of(v * L, L), L)
            for j in range(3):
                lab_v[j, sl] = lab_v[j, sl] * inv_l[j]
            return c

        lax.fori_loop(0, NCH, lab_scale, 0)

        def copies(slot, blk):
            off = base + blk * K
            return (
                pltpu.make_async_copy(
                    pvs_hbm.at[idx_v.at[pl.ds(blk * K, K)]],
                    pv_buf.at[slot], sems[slot]),
                pltpu.make_async_copy(
                    qv_hbm.at[pl.ds(off, K)], qv_buf.at[slot], sems[slot]),
                pltpu.make_async_copy(
                    qc_hbm.at[pl.ds(off, K)], qc_buf.at[slot], sems[slot]),
            )

        def issue(slot, blk):
            for c in copies(slot, blk):
                c.start()

        def wait(slot, blk):
            for c in copies(slot, blk):
                c.wait()

        def compute(slot, blk):
            pv_b, qv_b, qc_b = pv_buf.at[slot], qv_buf.at[slot], qc_buf.at[slot]

            def one_chunk(v, carry):
                sl = pl.ds(pl.multiple_of(v * L, L), L)
                w0, w1, w2 = qw_v[0, sl], qw_v[1, sl], qw_v[2, sl]
                l0, l1, l2 = lab_v[0, sl], lab_v[1, sl], lab_v[2, sl]
                nxt = []
                for r in range(K):
                    ss, d0, d1, d2 = carry[4 * r: 4 * r + 4]
                    x = qv_b[r, sl] * w0 + qc_b[r, sl] * w1 + pv_b[r, sl] * w2
                    t = 1.0 - 2.0 / (jnp.exp(x) + 1.0)
                    nxt += [ss + t * t, d0 + t * l0, d1 + t * l1, d2 + t * l2]
                return tuple(nxt)

            accs = lax.fori_loop(0, NCH, one_chunk, (zero,) * (4 * K))
            row0 = blk * K
            ssf = _reduce8([accs[4 * r + 0] for r in range(K)], lanes)
            inv_q = _vrsqrt(ssf)
            for j in range(N_LABELS):
                dj = _reduce8([accs[4 * r + 1 + j] for r in range(K)], lanes)
                out_v[pl.ds(j * (rpw + 8) + row0, L)] = dj * inv_q

        issue(0, 0)

        def outer(i2, c):
            b0 = i2 * 2
            issue(1, b0 + 1)
            wait(0, b0)
            compute(0, b0)

            @pl.when(b0 + 2 < nblk)
            def _():
                issue(0, b0 + 2)

            wait(1, b0 + 1)
            compute(1, b0 + 1)
            return c

        lax.fori_loop(0, nblk // 2, outer, 0)

        for j in range(N_LABELS):
            pltpu.sync_copy(
                out_v.at[pl.ds(j * (rpw + 8), rpw)],
                out_hbm.at[pl.ds(j * nrows + base, rpw)])

    return sc_fused


def _make_sc_gather(nrows):
    rpw = nrows // NW
    kg = 16
    nblk = rpw // kg

    @functools.partial(
        pl.kernel,
        out_type=jax.ShapeDtypeStruct((nrows, N_DIM), jnp.float32),
        mesh=_mesh,
        scratch_types=[
            pltpu.VMEM((rpw,), jnp.int32),
            pltpu.VMEM((2, kg, N_DIM), jnp.float32),
            pltpu.SemaphoreType.DMA,
            pltpu.SemaphoreType.DMA,
            pltpu.SemaphoreType.DMA,
            pltpu.SemaphoreType.DMA,
        ],
    )
    def sc_gather(pvs_hbm, idx_hbm, out_hbm, idx_v, buf,
                  gs0, gs1, ws0, ws1):
        wid = lax.axis_index("s") * NC + lax.axis_index("c")
        base = pl.multiple_of(wid * rpw, rpw)
        gsems = (gs0, gs1)
        wsems = (ws0, ws1)

        pltpu.sync_copy(idx_hbm.at[pl.ds(base, rpw)], idx_v)

        def g_copy(slot, blk):
            return pltpu.make_async_copy(
                pvs_hbm.at[idx_v.at[pl.ds(blk * kg, kg)]],
                buf.at[slot], gsems[slot])

        def w_copy(slot, blk):
            return pltpu.make_async_copy(
                buf.at[slot], out_hbm.at[pl.ds(base + blk * kg, kg)],
                wsems[slot])

        g_copy(0, 0).start()
        for b in range(nblk):
            s = b % 2
            if b + 1 < nblk:
                ns = (b + 1) % 2
                if b >= 1:
                    w_copy(ns, b - 1).wait()
                g_copy(ns, b + 1).start()
            g_copy(s, b).wait()
            w_copy(s, b).start()
        if nblk >= 2:
            w_copy((nblk - 2) % 2, nblk - 2).wait()
        w_copy((nblk - 1) % 2, nblk - 1).wait()

    return sc_gather


def _tc_dense_body(qv_ref, qc_ref, pv_ref, qw_ref, lab_ref, o_ref):
    w0 = qw_ref[0, :][None, :]
    w1 = qw_ref[1, :][None, :]
    w2 = qw_ref[2, :][None, :]
    x = qv_ref[...] * w0 + qc_ref[...] * w1 + pv_ref[...] * w2
    t = jnp.tanh(x)
    lab = lab_ref[...]
    ln = lab / (jnp.sqrt(jnp.sum(lab * lab, axis=1, keepdims=True)) + 1e-12)
    d = lax.dot_general(t, ln, (((1,), (1,)), ((), ())),
                        preferred_element_type=jnp.float32)
    nq = jnp.sqrt(jnp.sum(t * t, axis=1, keepdims=True)) + 1e-12
    o_ref[...] = d / nq


def _tc_dense(qv, qc, pv, qw, lab):
    grid = TC_ROWS // TC_BLOCK
    return pl.pallas_call(
        _tc_dense_body,
        grid=(grid,),
        in_specs=[
            pl.BlockSpec((TC_BLOCK, N_DIM), lambda i: (i, 0)),
            pl.BlockSpec((TC_BLOCK, N_DIM), lambda i: (i, 0)),
            pl.BlockSpec((TC_BLOCK, N_DIM), lambda i: (i, 0)),
            pl.BlockSpec((3, N_DIM), lambda i: (0, 0)),
            pl.BlockSpec((3, N_DIM), lambda i: (0, 0)),
        ],
        out_specs=pl.BlockSpec((TC_BLOCK, N_LABELS), lambda i: (i, 0)),
        out_shape=jax.ShapeDtypeStruct((TC_ROWS, N_LABELS), jnp.float32),
    )(qv, qc, pv, qw, lab)


_sc_fused = _make_sc_fused(SC_ROWS)
_sc_gather = _make_sc_gather(TC_ROWS)


def kernel(query_vec, qclass_vec, pvs, query_weight, label, product_idx):
    qw = query_weight.astype(jnp.float32)
    lab = label.astype(jnp.float32)
    idx = product_idx.astype(jnp.int32)

    s1 = _sc_fused(query_vec[:SC_ROWS], qclass_vec[:SC_ROWS], pvs,
                   qw, lab, idx[:SC_ROWS])
    pv2 = _sc_gather(pvs, idx[SC_ROWS:])
    s2 = _tc_dense(query_vec[SC_ROWS:], qclass_vec[SC_ROWS:], pv2, qw, lab)
    return jnp.concatenate(
        [s1.reshape(N_LABELS, SC_ROWS).T, s2], axis=0)


# speedup vs baseline: 1.1289x; 1.1289x over previous
import functools

import jax
import jax.numpy as jnp
from jax import lax
from jax.experimental import pallas as pl
from jax.experimental.pallas import tpu as pltpu
from jax.experimental.pallas import tpu_sc as plsc

N_PRODUCTS = 100000
N_DIM = 1024
BATCH = 16384
N_LABELS = 3

NC, NS, L = 2, 16, 16
NW = NC * NS
RPW = BATCH // NW
K = 8
NBLK = RPW // K
NCH = N_DIM // L


def _vrsqrt(x):
    i = lax.bitcast_convert_type(x, jnp.int32)
    magic = jnp.full((L,), 0x5F3759DF, jnp.int32)
    one = jnp.full((L,), 1, jnp.int32)
    y = lax.bitcast_convert_type(
        magic - lax.shift_right_arithmetic(i, one), jnp.float32)
    for _ in range(3):
        y = y * (1.5 - 0.5 * x * y * y)
    return y


def _lane_sum(x, lanes):
    for k in (8, 4, 2, 1):
        x = x + x.at[lanes ^ k].get(mode="promise_in_bounds")
    return x


def _shuf(x, lanes, k):
    return x.at[lanes ^ k].get(mode="promise_in_bounds")


def _at(x, idx):
    return x.at[idx].get(mode="promise_in_bounds")


def _merge(a, b, k, lanes):
    m = (lanes & k) != 0
    keep = jnp.where(m, b, a)
    give = jnp.where(m, a, b)
    return keep + _shuf(give, lanes, k)


def _reduce8(vs, lanes):
    for k in (1, 2, 4):
        vs = [_merge(vs[2 * i], vs[2 * i + 1], k, lanes)
              for i in range(len(vs) // 2)]
    z = vs[0]
    return z + _shuf(z, lanes, 8)


_mesh = plsc.VectorSubcoreMesh(
    core_axis_name="c", subcore_axis_name="s", num_cores=NC, num_subcores=NS
)


@functools.partial(
    pl.kernel,
    out_type=jax.ShapeDtypeStruct((N_LABELS * BATCH,), jnp.float32),
    mesh=_mesh,
    scratch_types=[
        pltpu.VMEM((RPW,), jnp.int32),
        pltpu.VMEM((3, N_DIM), jnp.float32),
        pltpu.VMEM((3, N_DIM), jnp.float32),
        pltpu.VMEM((2, K, N_DIM), jnp.float32),
        pltpu.VMEM((2, K, N_DIM), jnp.float32),
        pltpu.VMEM((2, K, N_DIM), jnp.float32),
        pltpu.VMEM((RPW * N_LABELS,), jnp.float32),
        pltpu.SemaphoreType.DMA,
        pltpu.SemaphoreType.DMA,
    ],
)
def _fastvss_sc(qv_hbm, qc_hbm, pvs_hbm, qw_hbm, lab_hbm, idx_hbm, out_hbm,
                idx_v, qw_v, lab_v, pv_buf, qv_buf, qc_buf, out_v,
                sem0, sem1):
    wid = lax.axis_index("s") * NC + lax.axis_index("c")
    base = pl.multiple_of(wid * RPW, RPW)
    sems = (sem0, sem1)

    pltpu.sync_copy(idx_hbm.at[pl.ds(base, RPW)], idx_v)
    pltpu.sync_copy(qw_hbm, qw_v)
    pltpu.sync_copy(lab_hbm, lab_v)

    zero = jnp.zeros((L,), jnp.float32)
    lanes = lax.iota(jnp.int32, L)

    def pre_body(v, carry):
        sl = pl.ds(pl.multiple_of(v * L, L), L)
        for j in range(3):
            qw_v[j, sl] = qw_v[j, sl] * 2.0
        l0, l1, l2 = lab_v[0, sl], lab_v[1, sl], lab_v[2, sl]
        a0, a1, a2 = carry
        return (a0 + l0 * l0, a1 + l1 * l1, a2 + l2 * l2)

    la = lax.fori_loop(0, NCH, pre_body, (zero, zero, zero))
    inv_l = [_vrsqrt(_lane_sum(a, lanes)) for a in la]

    def lab_scale(v, c):
        sl = pl.ds(pl.multiple_of(v * L, L), L)
        for j in range(3):
            lab_v[j, sl] = lab_v[j, sl] * inv_l[j]
        return c

    lax.fori_loop(0, NCH, lab_scale, 0)

    def copies(slot, blk):
        off = base + blk * K
        return (
            pltpu.make_async_copy(
                pvs_hbm.at[idx_v.at[pl.ds(blk * K, K)]], pv_buf.at[slot], sems[slot]),
            pltpu.make_async_copy(
                qv_hbm.at[pl.ds(off, K)], qv_buf.at[slot], sems[slot]),
            pltpu.make_async_copy(
                qc_hbm.at[pl.ds(off, K)], qc_buf.at[slot], sems[slot]),
        )

    def issue(slot, blk):
        for c in copies(slot, blk):
            c.start()

    def wait(slot, blk):
        for c in copies(slot, blk):
            c.wait()

    def compute(slot, blk):
        pv_b, qv_b, qc_b = pv_buf.at[slot], qv_buf.at[slot], qc_buf.at[slot]

        def one_chunk(v, carry):
            sl = pl.ds(pl.multiple_of(v * L, L), L)
            w0, w1, w2 = qw_v[0, sl], qw_v[1, sl], qw_v[2, sl]
            l0, l1, l2 = lab_v[0, sl], lab_v[1, sl], lab_v[2, sl]
            nxt = []
            for r in range(K):
                ss, d0, d1, d2 = carry[4 * r: 4 * r + 4]
                x = qv_b[r, sl] * w0 + qc_b[r, sl] * w1 + pv_b[r, sl] * w2
                t = 1.0 - 2.0 / (jnp.exp(x) + 1.0)
                nxt += [ss + t * t, d0 + t * l0, d1 + t * l1, d2 + t * l2]
            return tuple(nxt)

        accs = plsc.parallel_loop(
            0, NCH, 1, carry=(zero,) * (4 * K))(one_chunk)
        ssf = _reduce8([accs[4 * r + 0] for r in range(K)], lanes)
        inv_q = _vrsqrt(ssf)
        return tuple(
            _reduce8([accs[4 * r + 1 + j] for r in range(K)], lanes) * inv_q
            for j in range(N_LABELS))

    shift16 = jnp.full((L,), 16, jnp.int32)

    def assemble(pair, a, b):
        for k in range(N_LABELS):
            p16 = lanes + (k * 16)
            lr = lax.shift_right_logical(p16 * 21846, shift16)
            j3 = p16 - lr * 3
            pa = jnp.where(
                j3 == 0, _at(a[0], lr),
                jnp.where(j3 == 1, _at(a[1], lr), _at(a[2], lr)))
            pb = jnp.where(
                j3 == 0, _at(b[0], lr),
                jnp.where(j3 == 1, _at(b[1], lr), _at(b[2], lr)))
            out_v[pl.ds(pair * 48 + k * 16, L)] = jnp.where(lr < 8, pa, pb)

    issue(0, 0)

    def outer(i2, c):
        b0 = i2 * 2
        issue(1, b0 + 1)
        wait(0, b0)
        sa = compute(0, b0)

        @pl.when(b0 + 2 < NBLK)
        def _():
            issue(0, b0 + 2)

        wait(1, b0 + 1)
        sb = compute(1, b0 + 1)
        assemble(i2, sa, sb)
        return c

    lax.fori_loop(0, NBLK // 2, outer, 0)

    pltpu.sync_copy(
        out_v, out_hbm.at[pl.ds(base * N_LABELS, RPW * N_LABELS)])


def kernel(query_vec, qclass_vec, pvs, query_weight, label, product_idx):
    flat = _fastvss_sc(
        query_vec,
        qclass_vec,
        pvs,
        query_weight.astype(jnp.float32),
        label.astype(jnp.float32),
        product_idx.astype(jnp.int32),
    )
    return flat.reshape(BATCH, N_LABELS)
